# baseline (speedup 1.0000x reference)
import jax
import jax.numpy as jnp
from jax.experimental import pallas as pl
from jax.experimental.pallas import tpu as pltpu

K = 8192
D = 256
_NCH = 16
_CRW = K // _NCH


_DEPTH = 3


def _start_in(x_hbm, buf, in_sems, c):
    pltpu.make_async_copy(
        x_hbm.at[pl.ds(c * _CRW, _CRW)], buf.at[c], in_sems.at[c]).start()


def _body(x_hbm, o_hbm, buf, in_sems, out_sems):
    for c in range(_DEPTH):
        _start_in(x_hbm, buf, in_sems, c)
    for c in range(_NCH):
        pltpu.make_async_copy(
            x_hbm.at[pl.ds(c * _CRW, _CRW)], buf.at[c], in_sems.at[c]).wait()
        x = buf[c]
        sq = x * x
        ones = jnp.ones((D, 1), jnp.float32)
        s = jax.lax.dot_general(sq, ones, (((1,), (0,)), ((), ())),
                                preferred_element_type=jnp.float32)
        inv = 1.0 / jnp.maximum(jnp.sqrt(s), 1e-12)
        buf[c] = x * inv
        pltpu.make_async_copy(
            buf.at[c], o_hbm.at[pl.ds(c * _CRW, _CRW)], out_sems.at[c]).start()
        if c + _DEPTH < _NCH:
            _start_in(x_hbm, buf, in_sems, c + _DEPTH)
    for c in range(_NCH):
        pltpu.make_async_copy(
            buf.at[c], o_hbm.at[pl.ds(c * _CRW, _CRW)], out_sems.at[c]).wait()


def kernel(prototypes):
    return pl.pallas_call(
        _body,
        in_specs=[pl.BlockSpec(memory_space=pl.ANY)],
        out_specs=pl.BlockSpec(memory_space=pl.ANY),
        out_shape=jax.ShapeDtypeStruct((K, D), jnp.float32),
        scratch_shapes=[
            pltpu.VMEM((_NCH, _CRW, D), jnp.float32),
            pltpu.SemaphoreType.DMA((_NCH,)),
            pltpu.SemaphoreType.DMA((_NCH,)),
        ],
    )(prototypes)


# speedup vs baseline: 1.2853x; 1.2853x over previous
import jax
import jax.numpy as jnp
from jax.experimental import pallas as pl
from jax.experimental.pallas import tpu as pltpu

K = 8192
D = 256
_NCH = 16
_CRW = K // _NCH


_DEPTH = 4


def _start_in(x_hbm, buf, in_sems, c):
    pltpu.make_async_copy(
        x_hbm.at[pl.ds(c * _CRW, _CRW)], buf.at[c], in_sems.at[c]).start()


def _body(x_hbm, o_hbm, buf, in_sems, out_sems):
    for c in range(_DEPTH):
        _start_in(x_hbm, buf, in_sems, c)
    for c in range(_NCH):
        pltpu.make_async_copy(
            x_hbm.at[pl.ds(c * _CRW, _CRW)], buf.at[c], in_sems.at[c]).wait()
        if c + _DEPTH < _NCH:
            _start_in(x_hbm, buf, in_sems, c + _DEPTH)
        x = buf[c]
        sq = x * x
        ones = jnp.ones((D, 1), jnp.float32)
        s = jax.lax.dot_general(sq, ones, (((1,), (0,)), ((), ())),
                                preferred_element_type=jnp.float32)
        inv = 1.0 / jnp.maximum(jnp.sqrt(s), 1e-12)
        buf[c] = x * inv
        pltpu.make_async_copy(
            buf.at[c], o_hbm.at[pl.ds(c * _CRW, _CRW)], out_sems.at[c]).start()
    for c in range(_NCH):
        pltpu.make_async_copy(
            buf.at[c], o_hbm.at[pl.ds(c * _CRW, _CRW)], out_sems.at[c]).wait()


def kernel(prototypes):
    return pl.pallas_call(
        _body,
        in_specs=[pl.BlockSpec(memory_space=pl.ANY)],
        out_specs=pl.BlockSpec(memory_space=pl.ANY),
        out_shape=jax.ShapeDtypeStruct((K, D), jnp.float32),
        scratch_shapes=[
            pltpu.VMEM((_NCH, _CRW, D), jnp.float32),
            pltpu.SemaphoreType.DMA((_NCH,)),
            pltpu.SemaphoreType.DMA((_NCH,)),
        ],
    )(prototypes)


# speedup vs baseline: 1.4469x; 1.1257x over previous
import jax
import jax.numpy as jnp
from jax.experimental import pallas as pl
from jax.experimental.pallas import tpu as pltpu

K = 8192
D = 256
_NCH = 8
_CRW = K // _NCH


_DEPTH = 3


def _start_in(x_hbm, buf, in_sems, c):
    pltpu.make_async_copy(
        x_hbm.at[pl.ds(c * _CRW, _CRW)], buf.at[c], in_sems.at[c]).start()


def _body(x_hbm, o_hbm, buf, in_sems, out_sems):
    for c in range(_DEPTH):
        _start_in(x_hbm, buf, in_sems, c)
    for c in range(_NCH):
        pltpu.make_async_copy(
            x_hbm.at[pl.ds(c * _CRW, _CRW)], buf.at[c], in_sems.at[c]).wait()
        if c + _DEPTH < _NCH:
            _start_in(x_hbm, buf, in_sems, c + _DEPTH)
        x = buf[c]
        sq = x * x
        ones = jnp.ones((D, 1), jnp.float32)
        s = jax.lax.dot_general(sq, ones, (((1,), (0,)), ((), ())),
                                preferred_element_type=jnp.float32)
        inv = 1.0 / jnp.maximum(jnp.sqrt(s), 1e-12)
        buf[c] = x * inv
        pltpu.make_async_copy(
            buf.at[c], o_hbm.at[pl.ds(c * _CRW, _CRW)], out_sems.at[c]).start()
    for c in range(_NCH):
        pltpu.make_async_copy(
            buf.at[c], o_hbm.at[pl.ds(c * _CRW, _CRW)], out_sems.at[c]).wait()


def kernel(prototypes):
    return pl.pallas_call(
        _body,
        in_specs=[pl.BlockSpec(memory_space=pl.ANY)],
        out_specs=pl.BlockSpec(memory_space=pl.ANY),
        out_shape=jax.ShapeDtypeStruct((K, D), jnp.float32),
        scratch_shapes=[
            pltpu.VMEM((_NCH, _CRW, D), jnp.float32),
            pltpu.SemaphoreType.DMA((_NCH,)),
            pltpu.SemaphoreType.DMA((_NCH,)),
        ],
    )(prototypes)


# speedup vs baseline: 1.6392x; 1.1329x over previous
import jax
import jax.numpy as jnp
from jax.experimental import pallas as pl
from jax.experimental.pallas import tpu as pltpu

K = 8192
D = 256
_NCH = 8
_CRW = K // _NCH


_DEPTH = 8


def _start_in(x_hbm, buf, in_sems, c):
    pltpu.make_async_copy(
        x_hbm.at[pl.ds(c * _CRW, _CRW)], buf.at[c], in_sems.at[c]).start()


def _body(x_hbm, o_hbm, buf, in_sems, out_sems):
    for c in range(_DEPTH):
        _start_in(x_hbm, buf, in_sems, c)
    for c in range(_NCH):
        pltpu.make_async_copy(
            x_hbm.at[pl.ds(c * _CRW, _CRW)], buf.at[c], in_sems.at[c]).wait()
        if c + _DEPTH < _NCH:
            _start_in(x_hbm, buf, in_sems, c + _DEPTH)
        x = buf[c]
        sq = x * x
        ones = jnp.ones((D, 1), jnp.float32)
        s = jax.lax.dot_general(sq, ones, (((1,), (0,)), ((), ())),
                                preferred_element_type=jnp.float32)
        inv = 1.0 / jnp.maximum(jnp.sqrt(s), 1e-12)
        buf[c] = x * inv
        pltpu.make_async_copy(
            buf.at[c], o_hbm.at[pl.ds(c * _CRW, _CRW)], out_sems.at[c]).start()
    for c in range(_NCH):
        pltpu.make_async_copy(
            buf.at[c], o_hbm.at[pl.ds(c * _CRW, _CRW)], out_sems.at[c]).wait()


def kernel(prototypes):
    return pl.pallas_call(
        _body,
        in_specs=[pl.BlockSpec(memory_space=pl.ANY)],
        out_specs=pl.BlockSpec(memory_space=pl.ANY),
        out_shape=jax.ShapeDtypeStruct((K, D), jnp.float32),
        scratch_shapes=[
            pltpu.VMEM((_NCH, _CRW, D), jnp.float32),
            pltpu.SemaphoreType.DMA((_NCH,)),
            pltpu.SemaphoreType.DMA((_NCH,)),
        ],
    )(prototypes)


# speedup vs baseline: 1.7604x; 1.0740x over previous
import jax
import jax.numpy as jnp
from jax.experimental import pallas as pl
from jax.experimental.pallas import tpu as pltpu

K = 8192
D = 256
_SIZES = (2048, 2048, 1536, 1024, 1024, 512)
_OFFS = tuple(sum(_SIZES[:i]) for i in range(len(_SIZES)))
_NCH = len(_SIZES)


def _body(x_hbm, o_hbm, buf, in_sems, out_sems):
    def in_copy(c):
        return pltpu.make_async_copy(
            x_hbm.at[pl.ds(_OFFS[c], _SIZES[c])],
            buf.at[pl.ds(_OFFS[c], _SIZES[c])], in_sems.at[c])

    def out_copy(c):
        return pltpu.make_async_copy(
            buf.at[pl.ds(_OFFS[c], _SIZES[c])],
            o_hbm.at[pl.ds(_OFFS[c], _SIZES[c])], out_sems.at[c])

    for c in range(_NCH):
        in_copy(c).start()
    for c in range(_NCH):
        in_copy(c).wait()
        x = buf[pl.ds(_OFFS[c], _SIZES[c])]
        sq = x * x
        ones = jnp.ones((D, 1), jnp.float32)
        s = jax.lax.dot_general(sq, ones, (((1,), (0,)), ((), ())),
                                preferred_element_type=jnp.float32)
        inv = 1.0 / jnp.maximum(jnp.sqrt(s), 1e-12)
        buf[pl.ds(_OFFS[c], _SIZES[c])] = x * inv
        out_copy(c).start()
    for c in range(_NCH):
        out_copy(c).wait()


def kernel(prototypes):
    return pl.pallas_call(
        _body,
        in_specs=[pl.BlockSpec(memory_space=pl.ANY)],
        out_specs=pl.BlockSpec(memory_space=pl.ANY),
        out_shape=jax.ShapeDtypeStruct((K, D), jnp.float32),
        scratch_shapes=[
            pltpu.VMEM((K, D), jnp.float32),
            pltpu.SemaphoreType.DMA((_NCH,)),
            pltpu.SemaphoreType.DMA((_NCH,)),
        ],
    )(prototypes)


# speedup vs baseline: 1.7841x; 1.0135x over previous
import jax
import jax.numpy as jnp
from jax.experimental import pallas as pl
from jax.experimental.pallas import tpu as pltpu

K = 8192
D = 256
_SIZES = (256, 2048, 2048, 2048, 1536, 256)
_OFFS = tuple(sum(_SIZES[:i]) for i in range(len(_SIZES)))
_NCH = len(_SIZES)


def _body(x_hbm, o_hbm, buf, in_sems, out_sems):
    def in_copy(c):
        return pltpu.make_async_copy(
            x_hbm.at[pl.ds(_OFFS[c], _SIZES[c])],
            buf.at[pl.ds(_OFFS[c], _SIZES[c])], in_sems.at[c])

    def out_copy(c):
        return pltpu.make_async_copy(
            buf.at[pl.ds(_OFFS[c], _SIZES[c])],
            o_hbm.at[pl.ds(_OFFS[c], _SIZES[c])], out_sems.at[c])

    for c in range(_NCH):
        in_copy(c).start()
    for c in range(_NCH):
        in_copy(c).wait()
        x = buf[pl.ds(_OFFS[c], _SIZES[c])]
        sq = x * x
        ones = jnp.ones((D, 1), jnp.float32)
        s = jax.lax.dot_general(sq, ones, (((1,), (0,)), ((), ())),
                                preferred_element_type=jnp.float32)
        inv = 1.0 / jnp.maximum(jnp.sqrt(s), 1e-12)
        buf[pl.ds(_OFFS[c], _SIZES[c])] = x * inv
        out_copy(c).start()
    for c in range(_NCH):
        out_copy(c).wait()


def kernel(prototypes):
    return pl.pallas_call(
        _body,
        in_specs=[pl.BlockSpec(memory_space=pl.ANY)],
        out_specs=pl.BlockSpec(memory_space=pl.ANY),
        out_shape=jax.ShapeDtypeStruct((K, D), jnp.float32),
        scratch_shapes=[
            pltpu.VMEM((K, D), jnp.float32),
            pltpu.SemaphoreType.DMA((_NCH,)),
            pltpu.SemaphoreType.DMA((_NCH,)),
        ],
    )(prototypes)


# speedup vs baseline: 1.7910x; 1.0038x over previous
import jax
import jax.numpy as jnp
from jax.experimental import pallas as pl
from jax.experimental.pallas import tpu as pltpu

K = 8192
D = 256
_SIZES = (256, 512, 1536, 2048, 2048, 1536, 256)
_OFFS = tuple(sum(_SIZES[:i]) for i in range(len(_SIZES)))
_NCH = len(_SIZES)


def _body(x_hbm, o_hbm, buf, in_sems, out_sems):
    def in_copy(c):
        return pltpu.make_async_copy(
            x_hbm.at[pl.ds(_OFFS[c], _SIZES[c])],
            buf.at[pl.ds(_OFFS[c], _SIZES[c])], in_sems.at[c])

    def out_copy(c):
        return pltpu.make_async_copy(
            buf.at[pl.ds(_OFFS[c], _SIZES[c])],
            o_hbm.at[pl.ds(_OFFS[c], _SIZES[c])], out_sems.at[c])

    for c in range(_NCH):
        in_copy(c).start()
    for c in range(_NCH):
        in_copy(c).wait()
        x = buf[pl.ds(_OFFS[c], _SIZES[c])]
        sq = x * x
        ones = jnp.ones((D, 1), jnp.float32)
        s = jax.lax.dot_general(sq, ones, (((1,), (0,)), ((), ())),
                                preferred_element_type=jnp.float32)
        inv = 1.0 / jnp.maximum(jnp.sqrt(s), 1e-12)
        buf[pl.ds(_OFFS[c], _SIZES[c])] = x * inv
        out_copy(c).start()
    for c in range(_NCH):
        out_copy(c).wait()


def kernel(prototypes):
    return pl.pallas_call(
        _body,
        in_specs=[pl.BlockSpec(memory_space=pl.ANY)],
        out_specs=pl.BlockSpec(memory_space=pl.ANY),
        out_shape=jax.ShapeDtypeStruct((K, D), jnp.float32),
        scratch_shapes=[
            pltpu.VMEM((K, D), jnp.float32),
            pltpu.SemaphoreType.DMA((_NCH,)),
            pltpu.SemaphoreType.DMA((_NCH,)),
        ],
    )(prototypes)
